# baseline (speedup 1.0000x reference)
import functools

import jax
import jax.numpy as jnp
from jax import lax
from jax.experimental import pallas as pl
from jax.experimental.pallas import tpu as pltpu
from jax.experimental.pallas import tpu_sc as plsc

MIN_RATING = 1.0
MAX_RATING = 5.0

B = 16384
D = 32
NC = 2
NS = 16
NW = NC * NS
BPW = B // NW
HALF = BPW // 2
CHUNK = 128
NCH = HALF // CHUNK
L = 16
NG = HALF // L

_mesh = plsc.VectorSubcoreMesh(core_axis_name="c", subcore_axis_name="s")


@functools.partial(
    pl.kernel,
    out_type=jax.ShapeDtypeStruct((B,), jnp.float32),
    mesh=_mesh,
    scratch_types=[
        pltpu.VMEM((BPW,), jnp.int32),
        pltpu.VMEM((BPW,), jnp.int32),
        pltpu.VMEM((BPW,), jnp.int32),
        pltpu.VMEM((BPW,), jnp.int32),
        pltpu.VMEM((HALF, CHUNK), jnp.float32),
        pltpu.VMEM((HALF, CHUNK), jnp.float32),
        pltpu.VMEM((BPW,), jnp.float32),
        pltpu.VMEM((BPW,), jnp.float32),
        pltpu.VMEM((BPW,), jnp.float32),
        pltpu.SemaphoreType.DMA,
    ],
    compiler_params=pltpu.CompilerParams(
        needs_layout_passes=False, use_tc_tiling_on_sc=True),
)
def _lmf_sc(uidx_hbm, midx_hbm, uw_hbm, ub_hbm, mw_hbm, mb_hbm, out_hbm,
            uidx_v, midx_v, urow_v, mrow_v, uw_v, mw_v, ub_v, mb_v, out_v,
            sem):
    wid = lax.axis_index("s") * NC + lax.axis_index("c")
    base = wid * BPW

    pltpu.sync_copy(uidx_hbm.at[pl.ds(base, BPW)], uidx_v)
    pltpu.sync_copy(midx_hbm.at[pl.ds(base, BPW)], midx_v)

    def shift_rows(q, carry):
        sl = pl.ds(q * L, L)
        urow_v[sl] = jax.lax.shift_right_logical(uidx_v[sl], 2)
        mrow_v[sl] = jax.lax.shift_right_logical(midx_v[sl], 2)
        return carry
    lax.fori_loop(0, BPW // L, shift_rows, None)

    bias_copies = []
    for j in range(4):
        sl = pl.ds(j * CHUNK, CHUNK)
        bias_copies.append(pltpu.async_copy(ub_hbm.at[uidx_v.at[sl]], ub_v.at[sl], sem))
        bias_copies.append(pltpu.async_copy(mb_hbm.at[midx_v.at[sl]], mb_v.at[sl], sem))

    def half(h):
        new_copies = []
        for j in range(NCH):
            isl = pl.ds((h * NCH + j) * CHUNK, CHUNK)
            sl = pl.ds(j * CHUNK, CHUNK)
            new_copies.append(pltpu.async_copy(uw_hbm.at[urow_v.at[isl]], uw_v.at[sl], sem))
            new_copies.append(pltpu.async_copy(mw_hbm.at[mrow_v.at[isl]], mw_v.at[sl], sem))
        for c in new_copies:
            c.wait()

        def group(g, carry):
            rows = lax.iota(jnp.int32, L) + g * L
            sl = pl.ds(h * HALF + g * L, L)
            ucol0 = (uidx_v[sl] & 3) * D
            mcol0 = (midx_v[sl] & 3) * D
            acc = ub_v[sl] + mb_v[sl]
            for j in range(D):
                u = plsc.load_gather(uw_v, [rows, ucol0 + j])
                m = plsc.load_gather(mw_v, [rows, mcol0 + j])
                acc = acc + u * m
            y = 1.0 / (1.0 + jnp.exp(-acc))
            out_v[sl] = y * (MAX_RATING - MIN_RATING) + MIN_RATING
            return carry

        lax.fori_loop(0, NG, group, None)

    for c in bias_copies:
        c.wait()
    for h in range(2):
        half(h)

    pltpu.sync_copy(out_v, out_hbm.at[pl.ds(base, BPW)])


def kernel(users, movies, user_weights, user_bias, movie_weights, movie_bias):
    uidx = users.reshape(-1).astype(jnp.int32)
    midx = movies.reshape(-1).astype(jnp.int32)
    uw4 = user_weights.reshape(user_weights.shape[0] // 4, 4 * D)
    mw4 = movie_weights.reshape(movie_weights.shape[0] // 4, 4 * D)
    out = _lmf_sc(uidx, midx, uw4, user_bias.reshape(-1),
                  mw4, movie_bias.reshape(-1))
    return out.reshape(B, 1)


# speedup vs baseline: 2.5193x; 2.5193x over previous
import functools

import jax
import jax.numpy as jnp
from jax import lax
from jax.experimental import pallas as pl
from jax.experimental.pallas import tpu as pltpu
from jax.experimental.pallas import tpu_sc as plsc

MIN_RATING = 1.0
MAX_RATING = 5.0

B = 16384
D = 32
NC = 2
NS = 16
NW = NC * NS
BPW = B // NW
CHUNK = 128
L = 16
NG = BPW // L
RING = 8

_mesh = plsc.VectorSubcoreMesh(core_axis_name="c", subcore_axis_name="s")


@functools.partial(
    pl.kernel,
    out_type=jax.ShapeDtypeStruct((B,), jnp.float32),
    mesh=_mesh,
    scratch_types=[
        pltpu.VMEM((BPW,), jnp.int32),
        pltpu.VMEM((BPW,), jnp.int32),
        pltpu.VMEM((BPW,), jnp.int32),
        pltpu.VMEM((BPW,), jnp.int32),
        pltpu.VMEM((BPW,), jnp.int32),
        pltpu.SMEM((BPW,), jnp.int32),
        pltpu.SMEM((BPW,), jnp.int32),
        pltpu.VMEM((RING, D, CHUNK), jnp.float32),
        pltpu.VMEM((BPW * D,), jnp.float32),
        pltpu.VMEM((BPW, CHUNK), jnp.float32),
        pltpu.VMEM((BPW,), jnp.float32),
        pltpu.VMEM((BPW,), jnp.float32),
        pltpu.VMEM((BPW,), jnp.float32),
        pltpu.SemaphoreType.DMA((RING,)),
        pltpu.SemaphoreType.DMA,
    ],
    compiler_params=pltpu.CompilerParams(
        needs_layout_passes=False, use_tc_tiling_on_sc=True),
)
def _lmf_sc(uidx_hbm, midx_hbm, uwt_hbm, ub_hbm, mw_hbm, mb_hbm, out_hbm,
            uidx_v, midx_v, ugrp_v, ucol_v, mrow_v, ugrp_s, ucol_s,
            ubufs, urows, mw_v, ub_v, mb_v, out_v, rsem, sem):
    wid = lax.axis_index("s") * NC + lax.axis_index("c")
    base = wid * BPW

    pltpu.sync_copy(uidx_hbm.at[pl.ds(base, BPW)], uidx_v)
    pltpu.sync_copy(midx_hbm.at[pl.ds(base, BPW)], midx_v)

    def decomp(q, carry):
        sl = pl.ds(q * L, L)
        u = uidx_v[sl]
        ugrp_v[sl] = jax.lax.shift_right_logical(u, 7)
        ucol_v[sl] = u & (CHUNK - 1)
        mrow_v[sl] = jax.lax.shift_right_logical(midx_v[sl], 2)
        return carry
    lax.fori_loop(0, BPW // L, decomp, None)

    def to_smem(q, carry):
        gvec = ugrp_v[pl.ds(q * L, L)]
        cvec = ucol_v[pl.ds(q * L, L)]
        for l in range(L):
            ugrp_s[q * L + l] = gvec[l]
            ucol_s[q * L + l] = cvec[l]
        return carry
    lax.fori_loop(0, BPW // L, to_smem, None)

    copies = []
    for j in range(4):
        sl = pl.ds(j * CHUNK, CHUNK)
        copies.append(pltpu.async_copy(ub_hbm.at[uidx_v.at[sl]], ub_v.at[sl], sem))
        copies.append(pltpu.async_copy(mb_hbm.at[midx_v.at[sl]], mb_v.at[sl], sem))
        copies.append(pltpu.async_copy(mw_hbm.at[mrow_v.at[sl]],
                                       mw_v.at[pl.ds(j * CHUNK, CHUNK)], sem))

    def fire(p, slot):
        g = ugrp_s[p]
        off = pl.multiple_of(g * CHUNK, CHUNK)
        return pltpu.async_copy(uwt_hbm.at[:, pl.ds(off, CHUNK)],
                                ubufs.at[slot], rsem.at[slot])

    for p0 in range(RING):
        fire(p0, p0)

    jrow = lax.iota(jnp.int32, L)

    def ring_step(p, carry):
        slot = lax.rem(p, RING)
        pltpu.make_async_copy(uwt_hbm.at[:, pl.ds(0, CHUNK)],
                              ubufs.at[slot], rsem.at[slot]).wait()
        s16 = jnp.full((L,), slot, jnp.int32)
        c16 = jnp.full((L,), ucol_s[p], jnp.int32)
        u_lo = plsc.load_gather(ubufs, [s16, jrow, c16])
        u_hi = plsc.load_gather(ubufs, [s16, jrow + L, c16])
        urows[pl.ds(p * D, L)] = u_lo
        urows[pl.ds(p * D + L, L)] = u_hi

        @pl.when(p + RING < BPW)
        def _():
            fire(p + RING, slot)
        return carry

    lax.fori_loop(0, BPW, ring_step, None)

    for c in copies:
        c.wait()

    def group(g, carry):
        rows = lax.iota(jnp.int32, L) + g * L
        sl = pl.ds(g * L, L)
        mcol0 = (midx_v[sl] & 3) * D
        acc = ub_v[sl] + mb_v[sl]
        for j in range(D):
            u = plsc.load_gather(urows, [rows * D + j])
            m = plsc.load_gather(mw_v, [rows, mcol0 + j])
            acc = acc + u * m
        y = 1.0 / (1.0 + jnp.exp(-acc))
        out_v[sl] = y * (MAX_RATING - MIN_RATING) + MIN_RATING
        return carry

    lax.fori_loop(0, NG, group, None)

    pltpu.sync_copy(out_v, out_hbm.at[pl.ds(base, BPW)])


def kernel(users, movies, user_weights, user_bias, movie_weights, movie_bias):
    uidx = users.reshape(-1).astype(jnp.int32)
    midx = movies.reshape(-1).astype(jnp.int32)
    mw4 = movie_weights.reshape(movie_weights.shape[0] // 4, 4 * D)
    out = _lmf_sc(uidx, midx, user_weights.T, user_bias.reshape(-1),
                  mw4, movie_bias.reshape(-1))
    return out.reshape(B, 1)


# speedup vs baseline: 2.7275x; 1.0826x over previous
import functools

import jax
import jax.numpy as jnp
from jax import lax
from jax.experimental import pallas as pl
from jax.experimental.pallas import tpu as pltpu
from jax.experimental.pallas import tpu_sc as plsc

MIN_RATING = 1.0
MAX_RATING = 5.0

B = 16384
D = 32
NC = 2
NS = 16
NW = NC * NS
BPW = B // NW
CHUNK = 128
L = 16
NG = BPW // L
RING = 8

_mesh = plsc.VectorSubcoreMesh(core_axis_name="c", subcore_axis_name="s")


@functools.partial(
    pl.kernel,
    out_type=jax.ShapeDtypeStruct((B,), jnp.float32),
    mesh=_mesh,
    scratch_types=[
        pltpu.VMEM((BPW,), jnp.int32),
        pltpu.VMEM((BPW,), jnp.int32),
        pltpu.VMEM((BPW,), jnp.int32),
        pltpu.VMEM((BPW,), jnp.int32),
        pltpu.VMEM((BPW,), jnp.int32),
        pltpu.SMEM((BPW,), jnp.int32),
        pltpu.SMEM((BPW,), jnp.int32),
        pltpu.VMEM((RING, D, CHUNK), jnp.float32),
        pltpu.VMEM((BPW * D,), jnp.float32),
        pltpu.VMEM((BPW, CHUNK), jnp.float32),
        pltpu.VMEM((BPW,), jnp.float32),
        pltpu.VMEM((BPW,), jnp.float32),
        pltpu.VMEM((BPW,), jnp.float32),
        pltpu.SemaphoreType.DMA((RING,)),
        pltpu.SemaphoreType.DMA,
    ],
    compiler_params=pltpu.CompilerParams(
        needs_layout_passes=False, use_tc_tiling_on_sc=True),
)
def _lmf_sc(uidx_hbm, midx_hbm, uwt_hbm, ub_hbm, mw_hbm, mb_hbm, out_hbm,
            uidx_v, midx_v, ugrp_v, ucol_v, mrow_v, ugrp_s, ucol_s,
            ubufs, urows, mw_v, ub_v, mb_v, out_v, rsem, sem):
    wid = lax.axis_index("s") * NC + lax.axis_index("c")
    base = wid * BPW

    pltpu.sync_copy(uidx_hbm.at[pl.ds(base, BPW)], uidx_v)
    pltpu.sync_copy(midx_hbm.at[pl.ds(base, BPW)], midx_v)

    def decomp(q, carry):
        sl = pl.ds(q * L, L)
        u = uidx_v[sl]
        ugrp_v[sl] = jax.lax.shift_right_logical(u, 7)
        ucol_v[sl] = u & (CHUNK - 1)
        mrow_v[sl] = jax.lax.shift_right_logical(midx_v[sl], 2)
        return carry
    lax.fori_loop(0, BPW // L, decomp, None)

    def to_smem(q, carry):
        gvec = ugrp_v[pl.ds(q * L, L)]
        cvec = ucol_v[pl.ds(q * L, L)]
        for l in range(L):
            ugrp_s[q * L + l] = gvec[l]
            ucol_s[q * L + l] = cvec[l]
        return carry
    lax.fori_loop(0, BPW // L, to_smem, None)

    copies = []
    for j in range(4):
        sl = pl.ds(j * CHUNK, CHUNK)
        copies.append(pltpu.async_copy(ub_hbm.at[uidx_v.at[sl]], ub_v.at[sl], sem))
        copies.append(pltpu.async_copy(mb_hbm.at[midx_v.at[sl]], mb_v.at[sl], sem))
        copies.append(pltpu.async_copy(mw_hbm.at[mrow_v.at[sl]],
                                       mw_v.at[pl.ds(j * CHUNK, CHUNK)], sem))

    def fire(p, slot):
        g = ugrp_s[p]
        off = pl.multiple_of(g * CHUNK, CHUNK)
        return pltpu.async_copy(uwt_hbm.at[:, pl.ds(off, CHUNK)],
                                ubufs.at[slot], rsem.at[slot])

    for p0 in range(RING):
        fire(p0, p0)

    jrow = lax.iota(jnp.int32, L)

    for c in copies:
        c.wait()

    def ring_step(p, carry):
        slot = lax.rem(p, RING)
        pltpu.make_async_copy(uwt_hbm.at[:, pl.ds(0, CHUNK)],
                              ubufs.at[slot], rsem.at[slot]).wait()
        s16 = jnp.full((L,), slot, jnp.int32)
        c16 = jnp.full((L,), ucol_s[p], jnp.int32)
        u_lo = plsc.load_gather(ubufs, [s16, jrow, c16])
        u_hi = plsc.load_gather(ubufs, [s16, jrow + L, c16])
        urows[pl.ds(p * D, L)] = u_lo
        urows[pl.ds(p * D + L, L)] = u_hi

        @pl.when(p + RING < BPW)
        def _():
            fire(p + RING, slot)

        @pl.when((p & (L - 1)) == (L - 1))
        def _():
            g = jax.lax.shift_right_logical(p, 4)
            rows = lax.iota(jnp.int32, L) + g * L
            sl = pl.ds(g * L, L)
            mcol0 = (midx_v[sl] & 3) * D
            acc = ub_v[sl] + mb_v[sl]
            for j in range(D):
                u = plsc.load_gather(urows, [rows * D + j])
                m = plsc.load_gather(mw_v, [rows, mcol0 + j])
                acc = acc + u * m
            y = 1.0 / (1.0 + jnp.exp(-acc))
            out_v[sl] = y * (MAX_RATING - MIN_RATING) + MIN_RATING
        return carry

    lax.fori_loop(0, BPW, ring_step, None)

    pltpu.sync_copy(out_v, out_hbm.at[pl.ds(base, BPW)])


def kernel(users, movies, user_weights, user_bias, movie_weights, movie_bias):
    uidx = users.reshape(-1).astype(jnp.int32)
    midx = movies.reshape(-1).astype(jnp.int32)
    mw4 = movie_weights.reshape(movie_weights.shape[0] // 4, 4 * D)
    out = _lmf_sc(uidx, midx, user_weights.T, user_bias.reshape(-1),
                  mw4, movie_bias.reshape(-1))
    return out.reshape(B, 1)
